# Initial kernel scaffold; baseline (speedup 1.0000x reference)
import jax
import jax.numpy as jnp
from jax.experimental import pallas as pl


def kernel(cls_preds, reg_preds, anchors, gt_boxes, gt_labels):
    raise NotImplementedError("write your pallas kernel here")


# speedup vs baseline: 8.1843x; 8.1843x over previous
import jax
import jax.numpy as jnp
from jax.experimental import pallas as pl
from jax.experimental.pallas import tpu as pltpu

_N = 20000
_NPAD = 20480
_C = 2048
_NCHUNK = _NPAD // _C
_M = 100
_MPAD = 128
_NCLS = 80
_NBINS = 16
_TOPK = 9
_STRIDE = 8.0


def _body(a_ref, gtb_ref, gtl_ref, cls_ref, reg_ref, out_ref, d2_ref):
    f32 = jnp.float32
    gx1 = gtb_ref[:, 0:1]
    gy1 = gtb_ref[:, 1:2]
    gx2 = gtb_ref[:, 2:3]
    gy2 = gtb_ref[:, 3:4]
    gtl = gtl_ref[:, 0:1]
    gcx = (gx1 + gx2) * 0.5
    gcy = (gy1 + gy2) * 0.5
    area_g = (gx2 - gx1) * (gy2 - gy1)
    m_iota = jax.lax.broadcasted_iota(jnp.int32, (_MPAD, 1), 0)
    m_valid = m_iota < _M

    def anchor_cols(i):
        sl = pl.ds(i * _C, _C)
        ax1 = a_ref[0:1, sl]
        ay1 = a_ref[1:2, sl]
        ax2 = a_ref[2:3, sl]
        ay2 = a_ref[3:4, sl]
        return ax1, ay1, ax2, ay2

    def iou_chunk(ax1, ay1, ax2, ay2):
        iw = jnp.maximum(jnp.minimum(ax2, gx2) - jnp.maximum(ax1, gx1), 0.0)
        ih = jnp.maximum(jnp.minimum(ay2, gy2) - jnp.maximum(ay1, gy1), 0.0)
        inter = iw * ih
        area_a = (ax2 - ax1) * (ay2 - ay1)
        union = jnp.maximum(area_a + area_g - inter, 1e-9)
        return inter / union

    def phase_a(i, _):
        ax1, ay1, ax2, ay2 = anchor_cols(i)
        acx = (ax1 + ax2) * 0.5
        acy = (ay1 + ay2) * 0.5
        dx = acx - gcx
        dy = acy - gcy
        d2_ref[:, pl.ds(i * _C, _C)] = dx * dx + dy * dy
        return 0

    jax.lax.fori_loop(0, _NCHUNK, phase_a, 0)

    def min_pass(_, t_prev):
        def chunk_min(i, mcur):
            d2 = d2_ref[:, pl.ds(i * _C, _C)]
            cand = jnp.where(d2 > t_prev, d2, jnp.inf)
            return jnp.minimum(mcur, jnp.min(cand, axis=1, keepdims=True))

        return jax.lax.fori_loop(0, _NCHUNK, chunk_min,
                                 jnp.full((_MPAD, 1), jnp.inf, f32))

    t9 = jax.lax.fori_loop(0, _TOPK, min_pass,
                           jnp.full((_MPAD, 1), -jnp.inf, f32))

    def sel_body(i, carry):
        s1, s2, cnt = carry
        d2 = d2_ref[:, pl.ds(i * _C, _C)]
        ax1, ay1, ax2, ay2 = anchor_cols(i)
        iou = iou_chunk(ax1, ay1, ax2, ay2)
        sel = (d2 <= t9).astype(f32)
        s1 = s1 + jnp.sum(sel * iou, axis=1, keepdims=True)
        s2 = s2 + jnp.sum(sel * iou * iou, axis=1, keepdims=True)
        cnt = cnt + jnp.sum(sel, axis=1, keepdims=True)
        return s1, s2, cnt

    zer = jnp.zeros((_MPAD, 1), f32)
    s1, s2, cnt = jax.lax.fori_loop(0, _NCHUNK, sel_body, (zer, zer, zer))
    cnt = jnp.maximum(cnt, 1.0)
    mean = s1 / cnt
    var = jnp.maximum(s2 - cnt * mean * mean, 0.0) / jnp.maximum(cnt - 1.0, 1.0)
    thresh = mean + jnp.sqrt(var)

    c_iota = jax.lax.broadcasted_iota(jnp.int32, (_NCLS, 1), 0)
    b_iota = jax.lax.broadcasted_iota(jnp.int32, (_NBINS, 1), 0)

    def loss_body(i, carry):
        cls_acc, reg_acc, npos = carry
        sl = pl.ds(i * _C, _C)
        ax1, ay1, ax2, ay2 = anchor_cols(i)
        acx = (ax1 + ax2) * 0.5
        acy = (ay1 + ay2) * 0.5
        iou = iou_chunk(ax1, ay1, ax2, ay2)
        cand = iou >= thresh
        inside = ((acx >= gx1) & (acx <= gx2) & (acy >= gy1) & (acy <= gy2))
        pos = cand & inside & m_valid
        matched = jnp.max(jnp.where(pos, m_iota, -1), axis=0, keepdims=True)
        posany = matched >= 0
        posf = posany.astype(f32)
        safe = jnp.maximum(matched, 0)
        eq = m_iota == safe
        eqf = eq.astype(f32)
        q = jnp.sum(eqf * iou, axis=0, keepdims=True) * posf
        label = jnp.sum(jnp.where(eq, gtl, 0), axis=0, keepdims=True)
        tbx1 = jnp.sum(eqf * gx1, axis=0, keepdims=True)
        tby1 = jnp.sum(eqf * gy1, axis=0, keepdims=True)
        tbx2 = jnp.sum(eqf * gx2, axis=0, keepdims=True)
        tby2 = jnp.sum(eqf * gy2, axis=0, keepdims=True)
        npos = npos + jnp.sum(posf)

        p = cls_ref[:, sl]
        base = jnp.sum(jnp.maximum(p, 0.0) + jnp.log1p(jnp.exp(-jnp.abs(p))))
        p_label = jnp.sum(jnp.where(c_iota == label, p, 0.0), axis=0,
                          keepdims=True)
        cls_acc = cls_acc + base - jnp.sum(q * p_label)

        x = reg_ref[:, sl]
        rmax = jnp.max(x, axis=0, keepdims=True)
        e = jnp.exp(x - rmax)
        sides = (acx - tbx1, acy - tby1, tbx2 - acx, tby2 - acy)
        dfl = jnp.zeros((1, _C), f32)
        for s in range(4):
            tgt = jnp.clip(sides[s] * (1.0 / _STRIDE), 0.0,
                           _NBINS - 1 - 1e-3)
            left = jnp.floor(tgt)
            lefti = left.astype(jnp.int32)
            w_r = tgt - left
            w_l = 1.0 - w_r
            xs = x[s * _NBINS:(s + 1) * _NBINS, :]
            es = e[s * _NBINS:(s + 1) * _NBINS, :]
            lse = jnp.log(jnp.sum(es, axis=0, keepdims=True)) + rmax
            xsel_l = jnp.sum(jnp.where(b_iota == lefti, xs, 0.0), axis=0,
                             keepdims=True)
            xsel_r = jnp.sum(jnp.where(b_iota == lefti + 1, xs, 0.0), axis=0,
                             keepdims=True)
            dfl = dfl + (lse - w_l * xsel_l - w_r * xsel_r)
        reg_acc = reg_acc + jnp.sum(dfl * posf)
        return cls_acc, reg_acc, npos

    cls_acc, reg_acc, npos = jax.lax.fori_loop(
        0, _NCHUNK, loss_body, (jnp.float32(0), jnp.float32(0),
                                jnp.float32(0)))
    np_ = jnp.maximum(npos, 1.0)
    loss = cls_acc / np_ + reg_acc / (np_ * 4.0)
    out_ref[...] = jnp.full((8, 128), loss, f32)


def kernel(cls_preds, reg_preds, anchors, gt_boxes, gt_labels):
    f32 = jnp.float32
    a_t = jnp.pad(anchors.astype(f32).T, ((0, 4), (0, _NPAD - _N)),
                  constant_values=1e6)
    cls_t = jnp.pad(cls_preds.astype(f32).T, ((0, 0), (0, _NPAD - _N)),
                    constant_values=-100.0)
    reg_t = jnp.pad(reg_preds.astype(f32).reshape(_N, 64).T,
                    ((0, 0), (0, _NPAD - _N)))
    gtb = jnp.zeros((_MPAD, 128), f32).at[:_M, 0:4].set(gt_boxes.astype(f32))
    gtl = jnp.zeros((_MPAD, 128), jnp.int32).at[:_M, 0].set(
        gt_labels.astype(jnp.int32))
    out = pl.pallas_call(
        _body,
        out_shape=jax.ShapeDtypeStruct((8, 128), f32),
        scratch_shapes=[pltpu.VMEM((_MPAD, _NPAD), f32)],
    )(a_t, gtb, gtl, cls_t, reg_t)
    return out[0, 0]


# speedup vs baseline: 9.4437x; 1.1539x over previous
import jax
import jax.numpy as jnp
from jax.experimental import pallas as pl
from jax.experimental.pallas import tpu as pltpu

_N = 20000
_NPAD = 20480
_C = 2048
_NCHUNK = _NPAD // _C
_M = 100
_MPAD = 104
_NCLS = 80
_NBINS = 16
_TOPK = 9
_STRIDE = 8.0


def _body(a_ref, gtb_ref, gtl_ref, cls_ref, reg_ref, out_ref, d2_ref,
          iou_ref):
    f32 = jnp.float32
    gx1 = gtb_ref[:, 0:1]
    gy1 = gtb_ref[:, 1:2]
    gx2 = gtb_ref[:, 2:3]
    gy2 = gtb_ref[:, 3:4]
    gtl = gtl_ref[:, 0:1]
    gcx = (gx1 + gx2) * 0.5
    gcy = (gy1 + gy2) * 0.5
    area_g = (gx2 - gx1) * (gy2 - gy1)
    m_iota = jax.lax.broadcasted_iota(jnp.int32, (_MPAD, 1), 0)
    m_valid = m_iota < _M

    def anchor_cols(i):
        sl = pl.ds(i * _C, _C)
        ax1 = a_ref[0:1, sl]
        ay1 = a_ref[1:2, sl]
        ax2 = a_ref[2:3, sl]
        ay2 = a_ref[3:4, sl]
        return ax1, ay1, ax2, ay2

    def iou_chunk(ax1, ay1, ax2, ay2):
        iw = jnp.maximum(jnp.minimum(ax2, gx2) - jnp.maximum(ax1, gx1), 0.0)
        ih = jnp.maximum(jnp.minimum(ay2, gy2) - jnp.maximum(ay1, gy1), 0.0)
        inter = iw * ih
        area_a = (ax2 - ax1) * (ay2 - ay1)
        union = jnp.maximum(area_a + area_g - inter, 1e-9)
        return inter / union

    def phase_a(i, m1):
        ax1, ay1, ax2, ay2 = anchor_cols(i)
        acx = (ax1 + ax2) * 0.5
        acy = (ay1 + ay2) * 0.5
        dx = acx - gcx
        dy = acy - gcy
        d2 = dx * dx + dy * dy
        d2_ref[:, pl.ds(i * _C, _C)] = d2
        iou_ref[:, pl.ds(i * _C, _C)] = iou_chunk(ax1, ay1, ax2, ay2)
        return jnp.minimum(m1, jnp.min(d2, axis=1, keepdims=True))

    t1 = jax.lax.fori_loop(0, _NCHUNK, phase_a,
                           jnp.full((_MPAD, 1), jnp.inf, f32))

    def min_pass(_, t_prev):
        def chunk_min(i, mcur):
            d2 = d2_ref[:, pl.ds(i * _C, _C)]
            cand = jnp.where(d2 > t_prev, d2, jnp.inf)
            return jnp.minimum(mcur, jnp.min(cand, axis=1, keepdims=True))

        return jax.lax.fori_loop(0, _NCHUNK, chunk_min,
                                 jnp.full((_MPAD, 1), jnp.inf, f32))

    t9 = jax.lax.fori_loop(0, _TOPK - 1, min_pass, t1)

    def sel_body(i, carry):
        s1, s2, cnt = carry
        d2 = d2_ref[:, pl.ds(i * _C, _C)]
        iou = iou_ref[:, pl.ds(i * _C, _C)]
        sel = (d2 <= t9).astype(f32)
        s1 = s1 + jnp.sum(sel * iou, axis=1, keepdims=True)
        s2 = s2 + jnp.sum(sel * iou * iou, axis=1, keepdims=True)
        cnt = cnt + jnp.sum(sel, axis=1, keepdims=True)
        return s1, s2, cnt

    zer = jnp.zeros((_MPAD, 1), f32)
    s1, s2, cnt = jax.lax.fori_loop(0, _NCHUNK, sel_body, (zer, zer, zer))
    cnt = jnp.maximum(cnt, 1.0)
    mean = s1 / cnt
    var = jnp.maximum(s2 - cnt * mean * mean, 0.0) / jnp.maximum(cnt - 1.0, 1.0)
    thresh = mean + jnp.sqrt(var)

    c_iota = jax.lax.broadcasted_iota(jnp.int32, (_NCLS, 1), 0)
    b_iota = jax.lax.broadcasted_iota(jnp.int32, (_NBINS, 1), 0)

    def loss_body(i, carry):
        cls_acc, reg_acc, npos = carry
        sl = pl.ds(i * _C, _C)
        ax1, ay1, ax2, ay2 = anchor_cols(i)
        acx = (ax1 + ax2) * 0.5
        acy = (ay1 + ay2) * 0.5
        iou = iou_ref[:, sl]
        cand = iou >= thresh
        inside = ((acx >= gx1) & (acx <= gx2) & (acy >= gy1) & (acy <= gy2))
        pos = cand & inside & m_valid
        matched = jnp.max(jnp.where(pos, m_iota, -1), axis=0, keepdims=True)
        posany = matched >= 0
        posf = posany.astype(f32)
        safe = jnp.maximum(matched, 0)
        eq = m_iota == safe
        eqf = eq.astype(f32)
        q = jnp.sum(eqf * iou, axis=0, keepdims=True) * posf
        label = jnp.sum(jnp.where(eq, gtl, 0), axis=0, keepdims=True)
        tbx1 = jnp.sum(eqf * gx1, axis=0, keepdims=True)
        tby1 = jnp.sum(eqf * gy1, axis=0, keepdims=True)
        tbx2 = jnp.sum(eqf * gx2, axis=0, keepdims=True)
        tby2 = jnp.sum(eqf * gy2, axis=0, keepdims=True)
        npos = npos + jnp.sum(posf)

        p = cls_ref[:, sl]
        base = jnp.sum(jnp.maximum(p, 0.0) + jnp.log1p(jnp.exp(-jnp.abs(p))))
        p_label = jnp.sum(jnp.where(c_iota == label, p, 0.0), axis=0,
                          keepdims=True)
        cls_acc = cls_acc + base - jnp.sum(q * p_label)

        x = reg_ref[:, sl]
        rmax = jnp.max(x, axis=0, keepdims=True)
        e = jnp.exp(x - rmax)
        sides = (acx - tbx1, acy - tby1, tbx2 - acx, tby2 - acy)
        dfl = jnp.zeros((1, _C), f32)
        for s in range(4):
            tgt = jnp.clip(sides[s] * (1.0 / _STRIDE), 0.0,
                           _NBINS - 1 - 1e-3)
            left = jnp.floor(tgt)
            lefti = left.astype(jnp.int32)
            w_r = tgt - left
            w_l = 1.0 - w_r
            xs = x[s * _NBINS:(s + 1) * _NBINS, :]
            es = e[s * _NBINS:(s + 1) * _NBINS, :]
            lse = jnp.log(jnp.sum(es, axis=0, keepdims=True)) + rmax
            xsel_l = jnp.sum(jnp.where(b_iota == lefti, xs, 0.0), axis=0,
                             keepdims=True)
            xsel_r = jnp.sum(jnp.where(b_iota == lefti + 1, xs, 0.0), axis=0,
                             keepdims=True)
            dfl = dfl + (lse - w_l * xsel_l - w_r * xsel_r)
        reg_acc = reg_acc + jnp.sum(dfl * posf)
        return cls_acc, reg_acc, npos

    cls_acc, reg_acc, npos = jax.lax.fori_loop(
        0, _NCHUNK, loss_body, (jnp.float32(0), jnp.float32(0),
                                jnp.float32(0)))
    np_ = jnp.maximum(npos, 1.0)
    loss = cls_acc / np_ + reg_acc / (np_ * 4.0)
    out_ref[...] = jnp.full((8, 128), loss, f32)


def kernel(cls_preds, reg_preds, anchors, gt_boxes, gt_labels):
    f32 = jnp.float32
    a_t = jnp.pad(anchors.astype(f32).T, ((0, 4), (0, _NPAD - _N)),
                  constant_values=1e6)
    cls_t = jnp.pad(cls_preds.astype(f32).T, ((0, 0), (0, _NPAD - _N)),
                    constant_values=-100.0)
    reg_t = jnp.pad(reg_preds.astype(f32).reshape(_N, 64).T,
                    ((0, 0), (0, _NPAD - _N)))
    gtb = jnp.zeros((_MPAD, 128), f32).at[:_M, 0:4].set(gt_boxes.astype(f32))
    gtl = jnp.zeros((_MPAD, 128), jnp.int32).at[:_M, 0].set(
        gt_labels.astype(jnp.int32))
    out = pl.pallas_call(
        _body,
        out_shape=jax.ShapeDtypeStruct((8, 128), f32),
        scratch_shapes=[pltpu.VMEM((_MPAD, _NPAD), f32),
                        pltpu.VMEM((_MPAD, _NPAD), f32)],
    )(a_t, gtb, gtl, cls_t, reg_t)
    return out[0, 0]


# speedup vs baseline: 9.5398x; 1.0102x over previous
import jax
import jax.numpy as jnp
from jax.experimental import pallas as pl
from jax.experimental.pallas import tpu as pltpu

_N = 20000
_NPAD = 20480
_C = 2048
_NCHUNK = _NPAD // _C
_M = 100
_MPAD = 104
_NCLS = 80
_NBINS = 16
_TOPK = 9
_STRIDE = 8.0


def _body(a_ref, gtb_ref, cls_ref, reg_ref, out_ref, d2_ref, iou_ref):
    f32 = jnp.float32
    gx1 = gtb_ref[:, 0:1]
    gy1 = gtb_ref[:, 1:2]
    gx2 = gtb_ref[:, 2:3]
    gy2 = gtb_ref[:, 3:4]
    gcx = (gx1 + gx2) * 0.5
    gcy = (gy1 + gy2) * 0.5
    area_g = (gx2 - gx1) * (gy2 - gy1)
    m_iota = jax.lax.broadcasted_iota(jnp.int32, (_MPAD, 1), 0)
    m_valid = m_iota < _M

    def anchor_cols(i):
        sl = pl.ds(i * _C, _C)
        ax1 = a_ref[0:1, sl]
        ay1 = a_ref[1:2, sl]
        ax2 = a_ref[2:3, sl]
        ay2 = a_ref[3:4, sl]
        return ax1, ay1, ax2, ay2

    def iou_chunk(ax1, ay1, ax2, ay2):
        iw = jnp.maximum(jnp.minimum(ax2, gx2) - jnp.maximum(ax1, gx1), 0.0)
        ih = jnp.maximum(jnp.minimum(ay2, gy2) - jnp.maximum(ay1, gy1), 0.0)
        inter = iw * ih
        area_a = (ax2 - ax1) * (ay2 - ay1)
        union = jnp.maximum(area_a + area_g - inter, 1e-9)
        return inter / union

    def phase_a(i, m1):
        ax1, ay1, ax2, ay2 = anchor_cols(i)
        acx = (ax1 + ax2) * 0.5
        acy = (ay1 + ay2) * 0.5
        dx = acx - gcx
        dy = acy - gcy
        d2 = dx * dx + dy * dy
        d2_ref[:, pl.ds(i * _C, _C)] = d2
        iou_ref[:, pl.ds(i * _C, _C)] = iou_chunk(ax1, ay1, ax2, ay2)
        return jnp.minimum(m1, jnp.min(d2, axis=1, keepdims=True))

    t1 = jax.lax.fori_loop(0, _NCHUNK, phase_a,
                           jnp.full((_MPAD, 1), jnp.inf, f32))

    def min_pass(_, t_prev):
        def chunk_min(i, mcur):
            d2 = d2_ref[:, pl.ds(i * _C, _C)]
            cand = jnp.where(d2 > t_prev, d2, jnp.inf)
            return jnp.minimum(mcur, jnp.min(cand, axis=1, keepdims=True))

        return jax.lax.fori_loop(0, _NCHUNK, chunk_min,
                                 jnp.full((_MPAD, 1), jnp.inf, f32))

    t9 = jax.lax.fori_loop(0, _TOPK - 1, min_pass, t1)

    def sel_body(i, carry):
        s1, s2, cnt = carry
        d2 = d2_ref[:, pl.ds(i * _C, _C)]
        iou = iou_ref[:, pl.ds(i * _C, _C)]
        sel = (d2 <= t9).astype(f32)
        s1 = s1 + jnp.sum(sel * iou, axis=1, keepdims=True)
        s2 = s2 + jnp.sum(sel * iou * iou, axis=1, keepdims=True)
        cnt = cnt + jnp.sum(sel, axis=1, keepdims=True)
        return s1, s2, cnt

    zer = jnp.zeros((_MPAD, 1), f32)
    s1, s2, cnt = jax.lax.fori_loop(0, _NCHUNK, sel_body, (zer, zer, zer))
    cnt = jnp.maximum(cnt, 1.0)
    mean = s1 / cnt
    var = jnp.maximum(s2 - cnt * mean * mean, 0.0) / jnp.maximum(cnt - 1.0, 1.0)
    thresh = jnp.where(m_valid, mean + jnp.sqrt(var), jnp.inf)

    c_iota = jax.lax.broadcasted_iota(jnp.int32, (_NCLS, 1), 0)
    b_iota = jax.lax.broadcasted_iota(jnp.int32, (_NBINS, 1), 0)
    gt5 = gtb_ref[:, 0:5]

    def loss_body(i, carry):
        cls_acc, reg_acc, npos = carry
        sl = pl.ds(i * _C, _C)
        ax1, ay1, ax2, ay2 = anchor_cols(i)
        acx = (ax1 + ax2) * 0.5
        acy = (ay1 + ay2) * 0.5
        iou = iou_ref[:, sl]
        inside = ((acx >= gx1) & (acx <= gx2) & (acy >= gy1) & (acy <= gy2))
        pos = (iou >= thresh) & inside
        matched = jnp.max(jnp.where(pos, m_iota, -1), axis=0, keepdims=True)
        posany = matched >= 0
        posf = posany.astype(f32)
        safe = jnp.maximum(matched, 0)
        eq = m_iota == safe
        eqf = eq.astype(f32)
        q = jnp.sum(eqf * iou, axis=0, keepdims=True) * posf
        gath = jax.lax.dot_general(gt5, eqf, (((0,), (0,)), ((), ())),
                                   preferred_element_type=f32)
        tbx1 = gath[0:1, :]
        tby1 = gath[1:2, :]
        tbx2 = gath[2:3, :]
        tby2 = gath[3:4, :]
        label = gath[4:5, :].astype(jnp.int32)
        npos = npos + jnp.sum(posf)

        p = cls_ref[:, sl]
        base = jnp.sum(jnp.maximum(p, 0.0) + jnp.log1p(jnp.exp(-jnp.abs(p))))
        p_label = jnp.sum(jnp.where(c_iota == label, p, 0.0), axis=0,
                          keepdims=True)
        cls_acc = cls_acc + base - jnp.sum(q * p_label)

        x = reg_ref[:, sl]
        rmax = jnp.max(x, axis=0, keepdims=True)
        e = jnp.exp(x - rmax)
        sides = (acx - tbx1, acy - tby1, tbx2 - acx, tby2 - acy)
        dfl = jnp.zeros((1, _C), f32)
        for s in range(4):
            tgt = jnp.clip(sides[s] * (1.0 / _STRIDE), 0.0,
                           _NBINS - 1 - 1e-3)
            left = jnp.floor(tgt)
            lefti = left.astype(jnp.int32)
            w_r = tgt - left
            w_l = 1.0 - w_r
            xs = x[s * _NBINS:(s + 1) * _NBINS, :]
            es = e[s * _NBINS:(s + 1) * _NBINS, :]
            lse = jnp.log(jnp.sum(es, axis=0, keepdims=True)) + rmax
            xsel_l = jnp.sum(jnp.where(b_iota == lefti, xs, 0.0), axis=0,
                             keepdims=True)
            xsel_r = jnp.sum(jnp.where(b_iota == lefti + 1, xs, 0.0), axis=0,
                             keepdims=True)
            dfl = dfl + (lse - w_l * xsel_l - w_r * xsel_r)
        reg_acc = reg_acc + jnp.sum(dfl * posf)
        return cls_acc, reg_acc, npos

    cls_acc, reg_acc, npos = jax.lax.fori_loop(
        0, _NCHUNK, loss_body, (jnp.float32(0), jnp.float32(0),
                                jnp.float32(0)))
    np_ = jnp.maximum(npos, 1.0)
    loss = cls_acc / np_ + reg_acc / (np_ * 4.0)
    out_ref[...] = jnp.full((8, 128), loss, f32)


def kernel(cls_preds, reg_preds, anchors, gt_boxes, gt_labels):
    f32 = jnp.float32
    a_t = jnp.pad(anchors.astype(f32).T, ((0, 4), (0, _NPAD - _N)),
                  constant_values=1e6)
    cls_t = jnp.pad(cls_preds.astype(f32).T, ((0, 0), (0, _NPAD - _N)),
                    constant_values=-100.0)
    reg_t = jnp.pad(reg_preds.astype(f32).reshape(_N, 64).T,
                    ((0, 0), (0, _NPAD - _N)))
    gtb = (jnp.zeros((_MPAD, 128), f32)
           .at[:_M, 0:4].set(gt_boxes.astype(f32))
           .at[:_M, 4].set(gt_labels.astype(f32)))
    out = pl.pallas_call(
        _body,
        out_shape=jax.ShapeDtypeStruct((8, 128), f32),
        scratch_shapes=[pltpu.VMEM((_MPAD, _NPAD), f32),
                        pltpu.VMEM((_MPAD, _NPAD), f32)],
    )(a_t, gtb, cls_t, reg_t)
    return out[0, 0]
